# Initial kernel scaffold; baseline (speedup 1.0000x reference)
import jax
import jax.numpy as jnp
from jax.experimental import pallas as pl


def kernel(obs):
    raise NotImplementedError("write your pallas kernel here")


# speedup vs baseline: 14.5758x; 14.5758x over previous
import jax
import jax.numpy as jnp
from jax import lax
from jax.experimental import pallas as pl
from jax.experimental.pallas import tpu as pltpu

HIST = 50
B_BLK = 256


def _hist_body(obs_ref, buf_ref, mask_ref):
    o = obs_ref[...]
    col = lax.broadcasted_iota(jnp.int32, o.shape, 1)
    zcol = (col < 6) | ((col >= 9) & (col < 12))
    m = jnp.where(zcol, 0.0, o)
    buf_ref[:, 0:HIST - 1, :] = jnp.broadcast_to(
        m[:, None, :], (o.shape[0], HIST - 1, o.shape[1]))
    buf_ref[:, HIST - 1:HIST, :] = o[:, None, :]
    mask_ref[...] = lax.broadcasted_iota(
        jnp.int32, (o.shape[0], HIST), 1) < (HIST - 1)


def kernel(obs):
    if obs.ndim == 1:
        obs = obs[:, None]
    B, D = obs.shape
    grid = (B // B_BLK,)
    buf, mask = pl.pallas_call(
        _hist_body,
        grid=grid,
        in_specs=[pl.BlockSpec((B_BLK, D), lambda i: (i, 0))],
        out_specs=[
            pl.BlockSpec((B_BLK, HIST, D), lambda i: (i, 0, 0)),
            pl.BlockSpec((B_BLK, HIST), lambda i: (i, 0)),
        ],
        out_shape=[
            jax.ShapeDtypeStruct((B, HIST, D), jnp.float32),
            jax.ShapeDtypeStruct((B, HIST), jnp.bool_),
        ],
    )(obs)
    return buf, mask
